# baseline (speedup 1.0000x reference)
import jax
import jax.numpy as jnp
from jax import lax
from jax.experimental import pallas as pl
from jax.experimental.pallas import tpu as pltpu
from jax.experimental.pallas import tpu_sc as plsc

L = 200
N = 1024
D = 64
R = L * N
NUM_CORES = 2
NUM_SUBCORES = 16
NW = NUM_CORES * NUM_SUBCORES
ROWS_PER_W = R // NW
CHUNK = 128
CHUNKS = ROWS_PER_W // CHUNK
PAIRS = CHUNKS // 2
LANES = 16
KG = D // LANES


def _sc_body(xt_hbm, st_hbm, tok_hbm, seg_hbm, pos_hbm, out_hbm,
             idx_all, segt_b, d_b, tok0, tok1, ob0, ob1, pos0, pos1,
             c00, c01, seg0, seg1, semg0, semg1, semo0, semo1):
    toks = (tok0, tok1)
    outs = (ob0, ob1)
    poss = (pos0, pos1)
    c0s = (c00, c01)
    segs = (seg0, seg1)
    semgs = (semg0, semg1)
    semos = (semo0, semo1)

    wid = lax.axis_index("s") * NUM_CORES + lax.axis_index("c")
    wbase = wid * ROWS_PER_W
    pltpu.sync_copy(xt_hbm.at[pl.ds(wbase, ROWS_PER_W)], idx_all)
    pltpu.sync_copy(seg_hbm, segt_b)
    for k in range(KG):
        ksl = pl.ds(k * LANES, LANES)
        d_b[0, ksl] = segt_b[1, ksl] - segt_b[0, ksl]

    def issue(g, s):
        base = wbase + g * CHUNK
        pos_row = base // N
        pltpu.async_copy(tok_hbm.at[idx_all.at[pl.ds(g * CHUNK, CHUNK)]],
                         toks[s], semgs[s])
        pltpu.async_copy(pos_hbm.at[pl.ds(pos_row, 1)], poss[s], semgs[s])
        pltpu.async_copy(st_hbm.at[pl.ds(base, CHUNK)], segs[s], semgs[s])

    def wait_gather(s):
        pltpu.make_async_copy(tok_hbm.at[idx_all.at[pl.ds(0, CHUNK)]],
                              toks[s], semgs[s]).wait()
        pltpu.make_async_copy(pos_hbm.at[pl.ds(0, 1)], poss[s],
                              semgs[s]).wait()
        pltpu.make_async_copy(st_hbm.at[pl.ds(0, CHUNK)], segs[s],
                              semgs[s]).wait()

    def wait_out(s):
        pltpu.make_async_copy(outs[s],
                              out_hbm.at[pl.ds(0, D), pl.ds(0, CHUNK)],
                              semos[s]).wait()

    iota = lax.iota(jnp.int32, LANES)

    def compute(s):
        tok = toks[s]
        ob = outs[s]
        c0 = c0s[s]
        for k in range(KG):
            ksl = pl.ds(k * LANES, LANES)
            c0[0, ksl] = poss[s][0, ksl] + segt_b[0, ksl]

        def ngrp_body(ng, carry):
            nvec = iota + ng * LANES
            nsl = pl.ds(ng * LANES, LANES)
            segf = segs[s][nsl].astype(jnp.float32)
            for kd in range(KG):
                c0v = c0[0, pl.ds(kd * LANES, LANES)]
                dbv = d_b[0, pl.ds(kd * LANES, LANES)]
                for jd in range(LANES):
                    d = kd * LANES + jd
                    dvec = jnp.full((LANES,), d, dtype=jnp.int32)
                    tokv = plsc.load_gather(tok, [nvec, dvec])
                    c0spl = jnp.full((LANES,), c0v[jd], dtype=jnp.float32)
                    dbspl = jnp.full((LANES,), dbv[jd], dtype=jnp.float32)
                    ob[d, nsl] = tokv + c0spl + segf * dbspl
            return carry

        lax.fori_loop(0, CHUNK // LANES, ngrp_body, 0)

    issue(0, 0)
    issue(1, 1)

    def pair_body(go, carry):
        a = 2 * go
        for s in (0, 1):
            g = a + s
            wait_gather(s)

            @pl.when(go > 0)
            def _():
                wait_out(s)

            compute(s)
            base = wbase + g * CHUNK
            l = base // N
            n0 = base - l * N
            pltpu.async_copy(outs[s],
                             out_hbm.at[pl.ds(l * D, D), pl.ds(n0, CHUNK)],
                             semos[s])

            @pl.when(go < PAIRS - 1)
            def _():
                issue(g + 2, s)

        return carry

    lax.fori_loop(0, PAIRS, pair_body, 0)
    wait_out(0)
    wait_out(1)


def kernel(x, segments, token_table, segment_table, pos_embedding):
    xt = jnp.transpose(x, (1, 0)).reshape(R).astype(jnp.int32)
    st = jnp.transpose(segments, (1, 0)).reshape(R).astype(jnp.int32)
    pos = pos_embedding[:, 0, :]
    mesh = plsc.VectorSubcoreMesh(core_axis_name="c", subcore_axis_name="s")
    out = pl.kernel(
        _sc_body,
        out_type=jax.ShapeDtypeStruct((L * D, N), jnp.float32),
        mesh=mesh,
        scratch_types=[
            pltpu.VMEM((ROWS_PER_W,), jnp.int32),
            pltpu.VMEM((2, D), jnp.float32),
            pltpu.VMEM((1, D), jnp.float32),
            pltpu.VMEM((CHUNK, D), jnp.float32),
            pltpu.VMEM((CHUNK, D), jnp.float32),
            pltpu.VMEM((D, CHUNK), jnp.float32),
            pltpu.VMEM((D, CHUNK), jnp.float32),
            pltpu.VMEM((1, D), jnp.float32),
            pltpu.VMEM((1, D), jnp.float32),
            pltpu.VMEM((1, D), jnp.float32),
            pltpu.VMEM((1, D), jnp.float32),
            pltpu.VMEM((CHUNK,), jnp.int32),
            pltpu.VMEM((CHUNK,), jnp.int32),
            pltpu.SemaphoreType.DMA,
            pltpu.SemaphoreType.DMA,
            pltpu.SemaphoreType.DMA,
            pltpu.SemaphoreType.DMA,
        ],
        compiler_params=pltpu.CompilerParams(use_tc_tiling_on_sc=False,
                                             needs_layout_passes=False),
    )(xt, st, token_table, segment_table, pos)
    return jnp.transpose(out.reshape(L, D, N), (0, 2, 1))


# speedup vs baseline: 1.1618x; 1.1618x over previous
import jax
import jax.numpy as jnp
from jax import lax
from jax.experimental import pallas as pl
from jax.experimental.pallas import tpu as pltpu
from jax.experimental.pallas import tpu_sc as plsc

L = 200
N = 1024
D = 64
R = L * N
NUM_CORES = 2
NUM_SUBCORES = 16
NW = NUM_CORES * NUM_SUBCORES
ROWS_PER_W = R // NW
CHUNK = 128
CHUNKS = ROWS_PER_W // CHUNK
PAIRS = CHUNKS // 2
LANES = 16
KG = D // LANES


def _sc_body(xt_hbm, st_hbm, tok_hbm, seg_hbm, pos_hbm, out_hbm,
             idx_all, segt_b, d_b, tok0, tok1, ob0, ob1, pos0, pos1,
             c00, c01, seg0, seg1, semg0, semg1, semo0, semo1):
    toks = (tok0, tok1)
    outs = (ob0, ob1)
    poss = (pos0, pos1)
    c0s = (c00, c01)
    segs = (seg0, seg1)
    semgs = (semg0, semg1)
    semos = (semo0, semo1)

    wid = lax.axis_index("s") * NUM_CORES + lax.axis_index("c")
    wbase = wid * ROWS_PER_W
    pltpu.sync_copy(xt_hbm.at[pl.ds(wbase, ROWS_PER_W)], idx_all)
    pltpu.sync_copy(seg_hbm, segt_b)
    for k in range(KG):
        ksl = pl.ds(k * LANES, LANES)
        d_b[0, ksl] = segt_b[1, ksl] - segt_b[0, ksl]

    def issue(g, s):
        base = wbase + g * CHUNK
        pos_row = base // N
        pltpu.async_copy(tok_hbm.at[idx_all.at[pl.ds(g * CHUNK, CHUNK)]],
                         toks[s], semgs[s])
        pltpu.async_copy(pos_hbm.at[pl.ds(pos_row, 1)], poss[s], semgs[s])
        pltpu.async_copy(st_hbm.at[pl.ds(base, CHUNK)], segs[s], semgs[s])

    def wait_gather(s):
        pltpu.make_async_copy(tok_hbm.at[idx_all.at[pl.ds(0, CHUNK)]],
                              toks[s], semgs[s]).wait()
        pltpu.make_async_copy(pos_hbm.at[pl.ds(0, 1)], poss[s],
                              semgs[s]).wait()
        pltpu.make_async_copy(st_hbm.at[pl.ds(0, CHUNK)], segs[s],
                              semgs[s]).wait()

    def wait_out(s):
        pltpu.make_async_copy(outs[s], out_hbm.at[pl.ds(0, CHUNK)],
                              semos[s]).wait()

    def compute(s):
        tok = toks[s]
        ob = outs[s]
        c0 = c0s[s]
        for k in range(KG):
            ksl = pl.ds(k * LANES, LANES)
            c0[0, ksl] = poss[s][0, ksl] + segt_b[0, ksl]

        def grp_body(gi, carry):
            rbase = gi * LANES
            svf = segs[s][pl.ds(rbase, LANES)].astype(jnp.float32)
            for j in range(LANES):
                spl = jnp.full((LANES,), svf[j], dtype=jnp.float32)
                r = rbase + j
                for k in range(KG):
                    ksl = pl.ds(k * LANES, LANES)
                    ob[r, ksl] = tok[r, ksl] + c0[0, ksl] + spl * d_b[0, ksl]
            return carry

        lax.fori_loop(0, CHUNK // LANES, grp_body, 0)

    issue(0, 0)
    issue(1, 1)

    def pair_body(go, carry):
        a = 2 * go
        for s in (0, 1):
            g = a + s
            wait_gather(s)

            @pl.when(go > 0)
            def _():
                wait_out(s)

            compute(s)
            pltpu.async_copy(outs[s],
                             out_hbm.at[pl.ds(wbase + g * CHUNK, CHUNK)],
                             semos[s])

            @pl.when(go < PAIRS - 1)
            def _():
                issue(g + 2, s)

        return carry

    lax.fori_loop(0, PAIRS, pair_body, 0)
    wait_out(0)
    wait_out(1)


def kernel(x, segments, token_table, segment_table, pos_embedding):
    xt = jnp.transpose(x, (1, 0)).reshape(R).astype(jnp.int32)
    st = jnp.transpose(segments, (1, 0)).reshape(R).astype(jnp.int32)
    pos = pos_embedding[:, 0, :]
    mesh = plsc.VectorSubcoreMesh(core_axis_name="c", subcore_axis_name="s")
    out = pl.kernel(
        _sc_body,
        out_type=jax.ShapeDtypeStruct((R, D), jnp.float32),
        mesh=mesh,
        scratch_types=[
            pltpu.VMEM((ROWS_PER_W,), jnp.int32),
            pltpu.VMEM((2, D), jnp.float32),
            pltpu.VMEM((1, D), jnp.float32),
            pltpu.VMEM((CHUNK, D), jnp.float32),
            pltpu.VMEM((CHUNK, D), jnp.float32),
            pltpu.VMEM((CHUNK, D), jnp.float32),
            pltpu.VMEM((CHUNK, D), jnp.float32),
            pltpu.VMEM((1, D), jnp.float32),
            pltpu.VMEM((1, D), jnp.float32),
            pltpu.VMEM((1, D), jnp.float32),
            pltpu.VMEM((1, D), jnp.float32),
            pltpu.VMEM((CHUNK,), jnp.int32),
            pltpu.VMEM((CHUNK,), jnp.int32),
            pltpu.SemaphoreType.DMA,
            pltpu.SemaphoreType.DMA,
            pltpu.SemaphoreType.DMA,
            pltpu.SemaphoreType.DMA,
        ],
        compiler_params=pltpu.CompilerParams(use_tc_tiling_on_sc=False),
    )(xt, st, token_table, segment_table, pos)
    return out.reshape(L, N, D)
